# baseline (speedup 1.0000x reference)
import functools

import jax
import jax.numpy as jnp
from jax.experimental import pallas as pl
from jax.experimental.pallas import tpu as pltpu


def _fused_block_kernel(xp_ref, d_ref, be_ref, w1_ref, b1_ref, w2_ref,
                        b2_ref, g_ref, o_ref, s_ref, *, K):
    H, W, C = o_ref.shape
    P = K // 2
    HW = H * W
    n_pairs = d_ref.shape[0]

    s_ref[...] = xp_ref[...].astype(jnp.bfloat16)

    taps = [(kh, kw) for kh in range(K) for kw in range(K)]
    acc = None
    for p in range(n_pairs):
        kha, kwa = taps[2 * p]
        khb, kwb = taps[min(2 * p + 1, K * K - 1)]
        wina = s_ref[kha:kha + H, kwa:kwa + W, :].reshape(HW, C)
        winb = s_ref[khb:khb + H, kwb:kwb + W, :].reshape(HW, C)
        lhs = jnp.concatenate([wina, winb], axis=1)
        d = jnp.dot(lhs, d_ref[p], preferred_element_type=jnp.float32)
        acc = d if acc is None else acc + d

    t = acc + be_ref[...]

    h = jnp.dot(t.astype(jnp.bfloat16), w1_ref[...],
                preferred_element_type=jnp.float32)
    h = h + b1_ref[...]
    h = h * jnp.clip(h + 3.0, 0.0, 6.0) * (1.0 / 6.0)
    y = jnp.dot(h.astype(jnp.bfloat16), w2_ref[...],
                preferred_element_type=jnp.float32)
    y = y + b2_ref[...]

    xres = xp_ref[P:P + H, P:P + W, :].reshape(HW, C)
    out = xres + g_ref[...] * y
    o_ref[...] = out.reshape(H, W, C).astype(o_ref.dtype)


def kernel(x, w_dw, b_dw, bn_w, bn_b, bn_mean, bn_var, w1, b1, w2, b2, gamma):
    N, C, H, W = x.shape
    K = w_dw.shape[-1]
    P = K // 2
    CE = w1.shape[1]
    Hp, Wp = H + 2 * P, W + 2 * P

    s = bn_w * jax.lax.rsqrt(bn_var + 1e-5)
    w_eff = jnp.transpose(w_dw[:, 0, :, :], (1, 2, 0)) * s
    b_eff = ((b_dw - bn_mean) * s + bn_b).reshape(1, C)

    wflat = w_eff.reshape(K * K, C)
    n_taps = K * K
    n_pairs = (n_taps + 1) // 2
    eye = jnp.eye(C, dtype=jnp.float32)
    diags = eye[None, :, :] * wflat[:, None, :]
    if n_taps % 2:
        diags = jnp.concatenate(
            [diags, jnp.zeros((1, C, C), jnp.float32)], axis=0)
    dmats = diags.reshape(n_pairs, 2 * C, C).astype(jnp.bfloat16)

    x_nhwc = jnp.transpose(x, (0, 2, 3, 1))
    x_pad = jnp.pad(x_nhwc, ((0, 0), (P, P), (P, P), (0, 0)))

    body = functools.partial(_fused_block_kernel, K=K)
    out_nhwc = pl.pallas_call(
        body,
        out_shape=jax.ShapeDtypeStruct((N, H, W, C), x.dtype),
        grid=(N,),
        in_specs=[
            pl.BlockSpec((None, Hp, Wp, C), lambda n: (n, 0, 0, 0)),
            pl.BlockSpec((n_pairs, 2 * C, C), lambda n: (0, 0, 0)),
            pl.BlockSpec((1, C), lambda n: (0, 0)),
            pl.BlockSpec((C, CE), lambda n: (0, 0)),
            pl.BlockSpec((1, CE), lambda n: (0, 0)),
            pl.BlockSpec((CE, C), lambda n: (0, 0)),
            pl.BlockSpec((1, C), lambda n: (0, 0)),
            pl.BlockSpec((1, C), lambda n: (0, 0)),
        ],
        out_specs=pl.BlockSpec((None, H, W, C), lambda n: (n, 0, 0, 0)),
        scratch_shapes=[pltpu.VMEM((Hp, Wp, C), jnp.bfloat16)],
        compiler_params=pltpu.CompilerParams(dimension_semantics=("parallel",)),
    )(x_pad, dmats, b_eff, w1.astype(jnp.bfloat16), b1.reshape(1, CE),
      w2.astype(jnp.bfloat16), b2.reshape(1, C), gamma.reshape(1, C))

    return jnp.transpose(out_nhwc, (0, 3, 1, 2))


# speedup vs baseline: 2.0739x; 2.0739x over previous
import functools

import jax
import jax.numpy as jnp
from jax.experimental import pallas as pl
from jax.experimental.pallas import tpu as pltpu


def _fused_block_kernel(xp_ref, w_ref, be_ref, w1_ref, b1_ref, w2_ref,
                        b2_ref, g_ref, o_ref, t_sc, xr_sc, *, N):
    H, W, C = o_ref.shape
    K = w_ref.shape[0]
    P = K // 2
    HW = H * W
    n = pl.program_id(0)

    @pl.when(n > 0)
    def _mlp():
        t = t_sc[...].astype(jnp.bfloat16)
        h = jnp.dot(t, w1_ref[...], preferred_element_type=jnp.float32)
        h = h + b1_ref[...]
        h = h * jnp.clip(h + 3.0, 0.0, 6.0)
        y = jnp.dot(h.astype(jnp.bfloat16), w2_ref[...],
                    preferred_element_type=jnp.float32)
        y = y + b2_ref[...]
        out = xr_sc[...] + g_ref[...] * y
        o_ref[...] = out.reshape(H, W, C).astype(o_ref.dtype)

    @pl.when(n < N)
    def _conv():
        acc = jnp.broadcast_to(be_ref[...].reshape(1, 1, C), (H, W, C))
        for kh in range(K):
            for kw in range(K):
                win = xp_ref[kh:kh + H, kw:kw + W, :]
                wv = w_ref[kh, kw:kw + 1, :].reshape(1, 1, C)
                acc = acc + win * wv
        t_sc[...] = acc.reshape(HW, C)
        xr_sc[...] = xp_ref[P:P + H, P:P + W, :].reshape(HW, C)


def kernel(x, w_dw, b_dw, bn_w, bn_b, bn_mean, bn_var, w1, b1, w2, b2, gamma):
    N, C, H, W = x.shape
    K = w_dw.shape[-1]
    P = K // 2
    CE = w1.shape[1]
    Hp, Wp = H + 2 * P, W + 2 * P

    s = bn_w * jax.lax.rsqrt(bn_var + 1e-5)
    w_eff = jnp.transpose(w_dw[:, 0, :, :], (1, 2, 0)) * s
    b_eff = ((b_dw - bn_mean) * s + bn_b).reshape(1, C)

    x_nhwc = jnp.transpose(x, (0, 2, 3, 1))
    x_pad = jnp.pad(x_nhwc, ((0, 0), (P, P), (P, P), (0, 0)))

    body = functools.partial(_fused_block_kernel, N=N)
    out_nhwc = pl.pallas_call(
        body,
        out_shape=jax.ShapeDtypeStruct((N, H, W, C), x.dtype),
        grid=(N + 1,),
        in_specs=[
            pl.BlockSpec((None, Hp, Wp, C),
                         lambda n: (jnp.minimum(n, N - 1), 0, 0, 0)),
            pl.BlockSpec((K, K, C), lambda n: (0, 0, 0)),
            pl.BlockSpec((1, C), lambda n: (0, 0)),
            pl.BlockSpec((C, CE), lambda n: (0, 0)),
            pl.BlockSpec((1, CE), lambda n: (0, 0)),
            pl.BlockSpec((CE, C), lambda n: (0, 0)),
            pl.BlockSpec((1, C), lambda n: (0, 0)),
            pl.BlockSpec((1, C), lambda n: (0, 0)),
        ],
        out_specs=pl.BlockSpec((None, H, W, C),
                               lambda n: (jnp.maximum(n - 1, 0), 0, 0, 0)),
        scratch_shapes=[pltpu.VMEM((H * W, C), jnp.float32),
                        pltpu.VMEM((H * W, C), jnp.float32)],
        compiler_params=pltpu.CompilerParams(
            dimension_semantics=("arbitrary",)),
    )(x_pad, w_eff, b_eff, w1.astype(jnp.bfloat16), b1.reshape(1, CE),
      (w2 * (1.0 / 6.0)).astype(jnp.bfloat16), b2.reshape(1, C),
      gamma.reshape(1, C))

    return jnp.transpose(out_nhwc, (0, 3, 1, 2))
